# Initial kernel scaffold; baseline (speedup 1.0000x reference)
import jax
import jax.numpy as jnp
from jax.experimental import pallas as pl


def kernel(pred, target, weight):
    raise NotImplementedError("write your pallas kernel here")


# speedup vs baseline: 4.0187x; 4.0187x over previous
import functools

import jax
import jax.numpy as jnp
from jax.experimental import pallas as pl
from jax.experimental.pallas import tpu as pltpu

_BINS = 30
_BM = 512


def _ghm_block(pred_ref, tgt_ref, w_ref, out_ref, acc_ref, *, tot):
    i = pl.program_id(0)
    nblk = pl.num_programs(0)

    @pl.when(i == 0)
    def _init():
        for b in range(_BINS):
            acc_ref[0, b] = 0.0
            acc_ref[1, b] = 0.0
        acc_ref[2, 0] = 0.0

    p = pred_ref[...]
    t = tgt_ref[...]
    w = w_ref[...]
    C = p.shape[1]

    col = jax.lax.broadcasted_iota(jnp.int32, p.shape, 1)
    onehot = col == t

    ap = jnp.abs(p)
    e = jnp.exp(-ap)
    r = 1.0 / (1.0 + e)
    s = jnp.where(p >= 0, r, e * r)
    g = jnp.where(onehot, 1.0 - s, s)
    loss = jnp.maximum(p, 0.0) + jnp.log1p(e) - jnp.where(onehot, p, 0.0)

    w_row = jnp.sum(jnp.where(onehot, w, 0.0), axis=1, keepdims=True)
    val = loss * w_row

    idx = jnp.clip(jnp.floor(g * _BINS), 0.0, _BINS - 1.0)

    for b in range(_BINS):
        m = idx == float(b)
        acc_ref[0, b] += jnp.sum(jnp.where(m, val, 0.0))
        acc_ref[1, b] += jnp.sum(jnp.where(m, 1.0, 0.0))
    acc_ref[2, 0] += jnp.sum(w_row)

    @pl.when(i == nblk - 1)
    def _fin():
        total = jnp.float32(0.0)
        n = jnp.float32(0.0)
        for b in range(_BINS):
            c = acc_ref[1, b]
            n += jnp.where(c > 0.0, 1.0, 0.0)
            total += acc_ref[0, b] / jnp.maximum(c, 1.0)
        wsum = acc_ref[2, 0] * C
        out_ref[0, 0] = (tot / n) * total / wsum


def kernel(pred, target, weight):
    B, C = pred.shape
    nblk = B // _BM
    t2 = target.reshape(B, 1)
    w2 = weight.reshape(1, C)
    out = pl.pallas_call(
        functools.partial(_ghm_block, tot=float(B * C)),
        grid=(nblk,),
        in_specs=[
            pl.BlockSpec((_BM, C), lambda i: (i, 0)),
            pl.BlockSpec((_BM, 1), lambda i: (i, 0)),
            pl.BlockSpec((1, C), lambda i: (0, 0)),
        ],
        out_specs=pl.BlockSpec(memory_space=pltpu.SMEM),
        out_shape=jax.ShapeDtypeStruct((1, 1), jnp.float32),
        scratch_shapes=[pltpu.SMEM((4, _BINS + 2), jnp.float32)],
    )(pred, t2, w2)
    return out[0, 0]


# speedup vs baseline: 4.0612x; 1.0106x over previous
import functools

import jax
import jax.numpy as jnp
from jax.experimental import pallas as pl
from jax.experimental.pallas import tpu as pltpu

_BINS = 30
_BM = 512


def _ghm_block(pred_ref, tgt_ref, w_ref, out_ref, acc_ref, *, tot):
    i = pl.program_id(0)
    nblk = pl.num_programs(0)

    @pl.when(i == 0)
    def _init():
        for k in range(_BINS):
            acc_ref[0, k] = 0.0
            acc_ref[1, k] = 0.0
        acc_ref[2, 0] = 0.0
        acc_ref[2, 1] = 0.0

    p = pred_ref[...]
    t = tgt_ref[...]
    w = w_ref[...]
    C = p.shape[1]

    col = jax.lax.broadcasted_iota(jnp.int32, p.shape, 1)
    onehot = col == t

    ap = jnp.abs(p)
    e = jnp.exp(-ap)
    r = 1.0 / (1.0 + e)
    s = jnp.where(p >= 0, r, e * r)
    g = jnp.where(onehot, 1.0 - s, s)
    loss = jnp.maximum(p, 0.0) + jnp.log1p(e) - jnp.where(onehot, p, 0.0)

    w_row = jnp.sum(jnp.where(onehot, w, 0.0), axis=1, keepdims=True)
    val = loss * w_row

    u = g * _BINS

    for k in range(1, _BINS):
        m = u < float(k)
        acc_ref[0, k] += jnp.sum(jnp.where(m, 1.0, 0.0))
        acc_ref[1, k] += jnp.sum(jnp.where(m, val, 0.0))
    acc_ref[2, 0] += jnp.sum(w_row)
    acc_ref[2, 1] += jnp.sum(val)

    @pl.when(i == nblk - 1)
    def _fin():
        n_elems = jnp.float32(tot)
        total = jnp.float32(0.0)
        n = jnp.float32(0.0)
        for b in range(_BINS):
            c_lo = acc_ref[0, b] if b > 0 else jnp.float32(0.0)
            c_hi = acc_ref[0, b + 1] if b + 1 < _BINS else n_elems
            s_lo = acc_ref[1, b] if b > 0 else jnp.float32(0.0)
            s_hi = acc_ref[1, b + 1] if b + 1 < _BINS else acc_ref[2, 1]
            cnt = c_hi - c_lo
            n += jnp.where(cnt > 0.0, 1.0, 0.0)
            total += (s_hi - s_lo) / jnp.maximum(cnt, 1.0)
        wsum = acc_ref[2, 0] * C
        out_ref[0, 0] = (tot / n) * total / wsum


def kernel(pred, target, weight):
    B, C = pred.shape
    nblk = B // _BM
    t2 = target.reshape(B, 1)
    w2 = weight.reshape(1, C)
    out = pl.pallas_call(
        functools.partial(_ghm_block, tot=float(B * C)),
        grid=(nblk,),
        in_specs=[
            pl.BlockSpec((_BM, C), lambda i: (i, 0)),
            pl.BlockSpec((_BM, 1), lambda i: (i, 0)),
            pl.BlockSpec((1, C), lambda i: (0, 0)),
        ],
        out_specs=pl.BlockSpec(memory_space=pltpu.SMEM),
        out_shape=jax.ShapeDtypeStruct((1, 1), jnp.float32),
        scratch_shapes=[pltpu.SMEM((4, _BINS + 2), jnp.float32)],
    )(pred, t2, w2)
    return out[0, 0]
